# baseline (speedup 1.0000x reference)
import functools

import jax
import jax.numpy as jnp
from jax import lax
from jax.experimental import pallas as pl
from jax.experimental.pallas import tpu as pltpu
from jax.experimental.pallas import tpu_sc as plsc

N = 10000
E = 320000
D = 128

NG = 8
NH = 2
NLOC = N // NH
CH = 128
NCHUNK = E // CH
UNROLL = 8


def _sc_segment_sums(hcols, gidx_exp, sidx16):
    mesh = plsc.VectorSubcoreMesh(core_axis_name="c", subcore_axis_name="s")

    @functools.partial(
        pl.kernel,
        mesh=mesh,
        out_type=[
            jax.ShapeDtypeStruct((2 * NG * N * 16,), jnp.float32),
            jax.ShapeDtypeStruct((2 * NH * NLOC,), jnp.float32),
        ],
        scratch_types=[
            pltpu.VMEM((NLOC * 16,), jnp.float32),
            pltpu.VMEM((NLOC + 16,), jnp.float32),
            pltpu.VMEM((CH, 16), jnp.float32),
            pltpu.VMEM((1, CH), jnp.int32),
            pltpu.VMEM((CH * 16,), jnp.int32),
            pltpu.SemaphoreType.DMA,
        ],
        compiler_params=pltpu.CompilerParams(
            needs_layout_passes=False,
            use_tc_tiling_on_sc=False,
        ),
    )
    def seg_kernel(hcols_hbm, gidx_hbm, sidx_hbm, out_hbm, ocnt_hbm,
                   accf, cnt, grows, gidx, sidxb, sem):
        c = lax.axis_index("c")
        s = lax.axis_index("s")
        g = s % NG
        half = s // NG

        zero16 = jnp.zeros((16,), jnp.float32)
        one16 = jnp.ones((16,), jnp.float32)
        iota16 = lax.iota(jnp.int32, 16)
        lane0 = iota16 == 0
        nloc16 = jnp.full((16,), NLOC, jnp.int32)
        zero16i = jnp.zeros((16,), jnp.int32)
        lo16 = jnp.full((16,), half * NLOC, jnp.int32)

        def zacc(i, carry):
            accf[pl.ds(i * 16, 16)] = zero16
            return carry
        lax.fori_loop(0, NLOC, zacc, 0)

        def zcnt(i, carry):
            cnt[pl.ds(i * 16, 16)] = zero16
            return carry
        lax.fori_loop(0, (NLOC + 16) // 16, zcnt, 0)

        gbase = (c * NG + g) * E
        sbase = c * E * 16

        def chunk(k, carry):
            off = k * CH
            pltpu.sync_copy(gidx_hbm.at[pl.ds(gbase + off, CH)], gidx.at[0])
            pltpu.sync_copy(sidx_hbm.at[pl.ds(sbase + off * 16, CH * 16)],
                            sidxb)
            pltpu.async_copy(hcols_hbm.at[gidx.at[0]], grows, sem).wait()

            def edge(eb, carry2):
                for u in range(UNROLL):
                    e = eb * UNROLL + u
                    bc = sidxb[pl.ds(e * 16, 16)]
                    rel = bc - lo16
                    msk = (rel >= zero16i) & (rel < nloc16)
                    li = rel * 16 + iota16
                    v = grows[e, :]
                    plsc.addupdate_scatter(accf, [li], v, mask=msk)
                    plsc.addupdate_scatter(cnt, [rel], one16,
                                           mask=msk & lane0)
                return carry2
            lax.fori_loop(0, CH // UNROLL, edge, 0)
            return carry
        lax.fori_loop(0, NCHUNK, chunk, 0)

        obase = ((c * NG + g) * N + half * NLOC) * 16
        pltpu.sync_copy(accf, out_hbm.at[pl.ds(obase, NLOC * 16)])

        @pl.when(g == 0)
        def _():
            cbase = (c * NH + half) * NLOC
            pltpu.sync_copy(cnt.at[pl.ds(0, NLOC)],
                            ocnt_hbm.at[pl.ds(cbase, NLOC)])

    return seg_kernel(hcols, gidx_exp, sidx16)


RBLK = 1000


def _mm_body(x_ref, w_ref, hc_ref):
    h = jnp.dot(x_ref[...], w_ref[...], preferred_element_type=jnp.float32)
    for g in range(NG):
        hc_ref[g] = h[:, 16 * g:16 * (g + 1)]


def _tc_matmul_cols(x, w):
    return pl.pallas_call(
        _mm_body,
        grid=(N // RBLK,),
        in_specs=[
            pl.BlockSpec((RBLK, D), lambda i: (i, 0)),
            pl.BlockSpec((D, D), lambda i: (0, 0)),
        ],
        out_specs=pl.BlockSpec((NG, RBLK, 16), lambda i: (0, i, 0)),
        out_shape=jax.ShapeDtypeStruct((NG, N, 16), jnp.float32),
    )(x, w)


def _comb_body(x_ref, m_ref, cb_ref, wr_ref, br_ref, wn_ref, x1_ref, hc_ref):
    pre = jnp.dot(x_ref[...], wr_ref[...], preferred_element_type=jnp.float32)
    m1 = m_ref[0] / jnp.maximum(cb_ref[0], 1.0)
    m2 = m_ref[1] / jnp.maximum(cb_ref[1], 1.0)
    x1 = jnp.maximum(pre + br_ref[...] + m1 + m2, 0.0)
    x1_ref[...] = x1
    h1 = jnp.dot(x1, wn_ref[...], preferred_element_type=jnp.float32)
    for g in range(NG):
        hc_ref[g] = h1[:, 16 * g:16 * (g + 1)]


def _tc_combine(x, sums, cnt_b, wr, br, wnext):
    return pl.pallas_call(
        _comb_body,
        grid=(N // RBLK,),
        in_specs=[
            pl.BlockSpec((RBLK, D), lambda i: (i, 0)),
            pl.BlockSpec((2, RBLK, D), lambda i: (0, i, 0)),
            pl.BlockSpec((2, RBLK, D), lambda i: (0, i, 0)),
            pl.BlockSpec((D, D), lambda i: (0, 0)),
            pl.BlockSpec((1, D), lambda i: (0, 0)),
            pl.BlockSpec((D, D), lambda i: (0, 0)),
        ],
        out_specs=[
            pl.BlockSpec((RBLK, D), lambda i: (i, 0)),
            pl.BlockSpec((NG, RBLK, 16), lambda i: (0, i, 0)),
        ],
        out_shape=[
            jax.ShapeDtypeStruct((N, D), jnp.float32),
            jax.ShapeDtypeStruct((NG, N, 16), jnp.float32),
        ],
    )(x, sums, cnt_b, wr, br.reshape(1, D), wnext)


def _final_body(x_ref, x1_ref, m_ref, cb_ref, wr_ref, br_ref,
                wfa_ref, wfb_ref, wfc_ref, bf_ref, o_ref):
    pre = jnp.dot(x1_ref[...], wr_ref[...], preferred_element_type=jnp.float32)
    m1 = m_ref[0] / jnp.maximum(cb_ref[0], 1.0)
    m2 = m_ref[1] / jnp.maximum(cb_ref[1], 1.0)
    x2 = jnp.maximum(pre + br_ref[...] + m1 + m2, 0.0)
    o = jnp.dot(x_ref[...], wfa_ref[...], preferred_element_type=jnp.float32)
    o += jnp.dot(x1_ref[...], wfb_ref[...], preferred_element_type=jnp.float32)
    o += jnp.dot(x2, wfc_ref[...], preferred_element_type=jnp.float32)
    o_ref[...] = o + bf_ref[...]


def _tc_final(x, x1, sums, cnt_b, wr, br, wf, bf):
    wfa, wfb, wfc = wf[:D], wf[D:2 * D], wf[2 * D:]
    return pl.pallas_call(
        _final_body,
        grid=(N // RBLK,),
        in_specs=[
            pl.BlockSpec((RBLK, D), lambda i: (i, 0)),
            pl.BlockSpec((RBLK, D), lambda i: (i, 0)),
            pl.BlockSpec((2, RBLK, D), lambda i: (0, i, 0)),
            pl.BlockSpec((2, RBLK, D), lambda i: (0, i, 0)),
            pl.BlockSpec((D, D), lambda i: (0, 0)),
            pl.BlockSpec((1, D), lambda i: (0, 0)),
            pl.BlockSpec((D, D), lambda i: (0, 0)),
            pl.BlockSpec((D, D), lambda i: (0, 0)),
            pl.BlockSpec((D, D), lambda i: (0, 0)),
            pl.BlockSpec((1, D), lambda i: (0, 0)),
        ],
        out_specs=pl.BlockSpec((RBLK, D), lambda i: (i, 0)),
        out_shape=jax.ShapeDtypeStruct((N, D), jnp.float32),
    )(x, x1, sums, cnt_b, wr, br.reshape(1, D), wfa, wfb, wfc,
      bf.reshape(1, D))


def _regroup(sums_flat):
    return (sums_flat.reshape(2, NG, N, 16)
            .transpose(0, 2, 1, 3)
            .reshape(2, N, D))


def _cnt_bcast(cnts_flat):
    return jnp.broadcast_to(cnts_flat.reshape(2, N, 1), (2, N, D))


def kernel(x, edge_index, edge_weight, W1_0, W2_0, Wr_0, br_0,
           W1_1, W2_1, Wr_1, br_1, Wf, bf):
    del edge_weight, W2_0, W2_1
    src = edge_index[0]
    dst = edge_index[1]
    goff = (jnp.arange(NG, dtype=jnp.int32) * N)[None, :, None]
    gidx_exp = (jnp.stack([src, dst])[:, None, :] + goff).reshape(-1)
    sidx16 = jnp.repeat(jnp.concatenate([dst, src]), 16)

    h0c = _tc_matmul_cols(x, W1_0)
    sums0, cnts0 = _sc_segment_sums(h0c.reshape(NG * N, 16), gidx_exp, sidx16)
    cnt_b = _cnt_bcast(cnts0)
    x1, h1c = _tc_combine(x, _regroup(sums0), cnt_b, Wr_0, br_0, W1_1)
    sums1, _ = _sc_segment_sums(h1c.reshape(NG * N, 16), gidx_exp, sidx16)
    out = _tc_final(x, x1, _regroup(sums1), cnt_b, Wr_1, br_1, Wf, bf)
    return out


# speedup vs baseline: 1.2527x; 1.2527x over previous
import functools

import jax
import jax.numpy as jnp
from jax import lax
from jax.experimental import pallas as pl
from jax.experimental.pallas import tpu as pltpu
from jax.experimental.pallas import tpu_sc as plsc

N = 10000
E = 320000
D = 128

NG = 8
NH = 2
NLOC = N // NH
CH = 128
NCHUNK = E // CH
UNROLL = 8


def _sc_segment_sums(hcols, gidx_exp, licid):
    mesh = plsc.VectorSubcoreMesh(core_axis_name="c", subcore_axis_name="s")
    REC = 2 * CH * 16

    @functools.partial(
        pl.kernel,
        mesh=mesh,
        out_type=[
            jax.ShapeDtypeStruct((2 * NG * N * 16,), jnp.float32),
            jax.ShapeDtypeStruct((2 * NH * NLOC,), jnp.float32),
        ],
        scratch_types=[
            pltpu.VMEM(((NLOC + 1) * 16,), jnp.float32),
            pltpu.VMEM((NLOC + 16,), jnp.float32),
            pltpu.VMEM((2, CH, 16), jnp.float32),
            pltpu.VMEM((2, CH), jnp.int32),
            pltpu.VMEM((2, REC), jnp.int32),
            pltpu.SemaphoreType.DMA,
            pltpu.SemaphoreType.DMA,
            pltpu.SemaphoreType.DMA,
            pltpu.SemaphoreType.DMA,
        ],
        compiler_params=pltpu.CompilerParams(
            needs_layout_passes=False,
            use_tc_tiling_on_sc=False,
        ),
    )
    def seg_kernel(hcols_hbm, gidx_hbm, licid_hbm, out_hbm, ocnt_hbm,
                   accf, cnt, grows, gidx, licb,
                   sem_g0, sem_g1, sem_i0, sem_i1):
        c = lax.axis_index("c")
        s = lax.axis_index("s")
        g = s % NG
        half = s // NG

        sem_g = (sem_g0, sem_g1)
        sem_i = (sem_i0, sem_i1)

        zero16 = jnp.zeros((16,), jnp.float32)
        one16 = jnp.ones((16,), jnp.float32)

        def zacc(i, carry):
            accf[pl.ds(i * 16, 16)] = zero16
            return carry
        lax.fori_loop(0, NLOC + 1, zacc, 0)

        def zcnt(i, carry):
            cnt[pl.ds(i * 16, 16)] = zero16
            return carry
        lax.fori_loop(0, (NLOC + 16) // 16, zcnt, 0)

        gbase = (c * NG + g) * E
        lbase = (c * NH + half) * NCHUNK * REC

        def issue_idx(k, b):
            pltpu.async_copy(gidx_hbm.at[pl.ds(gbase + k * CH, CH)],
                             gidx.at[b], sem_i[b])
            pltpu.async_copy(licid_hbm.at[pl.ds(lbase + k * REC, REC)],
                             licb.at[b], sem_i[b])

        def wait_idx(b):
            pltpu.make_async_copy(gidx_hbm.at[pl.ds(0, CH)],
                                  gidx.at[b], sem_i[b]).wait()
            pltpu.make_async_copy(licid_hbm.at[pl.ds(0, REC)],
                                  licb.at[b], sem_i[b]).wait()

        def issue_gather(b):
            pltpu.async_copy(hcols_hbm.at[gidx.at[b]], grows.at[b], sem_g[b])

        def wait_gather(b):
            pltpu.make_async_copy(hcols_hbm.at[pl.ds(0, CH)],
                                  grows.at[b], sem_g[b]).wait()

        issue_idx(0, 0)
        issue_idx(1, 1)
        wait_idx(0)
        issue_gather(0)

        def compute(b):
            def edge(eb, carry2):
                for u in range(UNROLL):
                    e = eb * UNROLL + u
                    li = licb[b, pl.ds(e * 16, 16)]
                    ci = licb[b, pl.ds(CH * 16 + e * 16, 16)]
                    v = grows[b, e, :]
                    plsc.addupdate_scatter(accf, [li], v)
                    plsc.addupdate_scatter(cnt, [ci], one16)
                return carry2
            lax.fori_loop(0, CH // UNROLL, edge, 0)

        def pipe(k2, carry):
            for b in range(2):
                kk = k2 * 2 + b
                nb = 1 - b
                wait_gather(b)

                @pl.when(kk < NCHUNK - 1)
                def _():
                    wait_idx(nb)
                    issue_gather(nb)

                compute(b)

                @pl.when(kk < NCHUNK - 2)
                def _():
                    issue_idx(kk + 2, b)
            return carry
        lax.fori_loop(0, NCHUNK // 2, pipe, 0)

        obase = ((c * NG + g) * N + half * NLOC) * 16
        pltpu.sync_copy(accf.at[pl.ds(0, NLOC * 16)],
                        out_hbm.at[pl.ds(obase, NLOC * 16)])

        @pl.when(g == 0)
        def _():
            cbase = (c * NH + half) * NLOC
            pltpu.sync_copy(cnt.at[pl.ds(0, NLOC)],
                            ocnt_hbm.at[pl.ds(cbase, NLOC)])

    return seg_kernel(hcols, gidx_exp, licid)


RBLK = 1000


def _mm_body(x_ref, w_ref, hc_ref):
    h = jnp.dot(x_ref[...], w_ref[...], preferred_element_type=jnp.float32)
    for g in range(NG):
        hc_ref[g] = h[:, 16 * g:16 * (g + 1)]


def _tc_matmul_cols(x, w):
    return pl.pallas_call(
        _mm_body,
        grid=(N // RBLK,),
        in_specs=[
            pl.BlockSpec((RBLK, D), lambda i: (i, 0)),
            pl.BlockSpec((D, D), lambda i: (0, 0)),
        ],
        out_specs=pl.BlockSpec((NG, RBLK, 16), lambda i: (0, i, 0)),
        out_shape=jax.ShapeDtypeStruct((NG, N, 16), jnp.float32),
    )(x, w)


def _comb_body(x_ref, m_ref, cb_ref, wr_ref, br_ref, wn_ref, x1_ref, hc_ref):
    pre = jnp.dot(x_ref[...], wr_ref[...], preferred_element_type=jnp.float32)
    m1 = m_ref[0] / jnp.maximum(cb_ref[0], 1.0)
    m2 = m_ref[1] / jnp.maximum(cb_ref[1], 1.0)
    x1 = jnp.maximum(pre + br_ref[...] + m1 + m2, 0.0)
    x1_ref[...] = x1
    h1 = jnp.dot(x1, wn_ref[...], preferred_element_type=jnp.float32)
    for g in range(NG):
        hc_ref[g] = h1[:, 16 * g:16 * (g + 1)]


def _tc_combine(x, sums, cnt_b, wr, br, wnext):
    return pl.pallas_call(
        _comb_body,
        grid=(N // RBLK,),
        in_specs=[
            pl.BlockSpec((RBLK, D), lambda i: (i, 0)),
            pl.BlockSpec((2, RBLK, D), lambda i: (0, i, 0)),
            pl.BlockSpec((2, RBLK, D), lambda i: (0, i, 0)),
            pl.BlockSpec((D, D), lambda i: (0, 0)),
            pl.BlockSpec((1, D), lambda i: (0, 0)),
            pl.BlockSpec((D, D), lambda i: (0, 0)),
        ],
        out_specs=[
            pl.BlockSpec((RBLK, D), lambda i: (i, 0)),
            pl.BlockSpec((NG, RBLK, 16), lambda i: (0, i, 0)),
        ],
        out_shape=[
            jax.ShapeDtypeStruct((N, D), jnp.float32),
            jax.ShapeDtypeStruct((NG, N, 16), jnp.float32),
        ],
    )(x, sums, cnt_b, wr, br.reshape(1, D), wnext)


def _final_body(x_ref, x1_ref, m_ref, cb_ref, wr_ref, br_ref,
                wfa_ref, wfb_ref, wfc_ref, bf_ref, o_ref):
    pre = jnp.dot(x1_ref[...], wr_ref[...], preferred_element_type=jnp.float32)
    m1 = m_ref[0] / jnp.maximum(cb_ref[0], 1.0)
    m2 = m_ref[1] / jnp.maximum(cb_ref[1], 1.0)
    x2 = jnp.maximum(pre + br_ref[...] + m1 + m2, 0.0)
    o = jnp.dot(x_ref[...], wfa_ref[...], preferred_element_type=jnp.float32)
    o += jnp.dot(x1_ref[...], wfb_ref[...], preferred_element_type=jnp.float32)
    o += jnp.dot(x2, wfc_ref[...], preferred_element_type=jnp.float32)
    o_ref[...] = o + bf_ref[...]


def _tc_final(x, x1, sums, cnt_b, wr, br, wf, bf):
    wfa, wfb, wfc = wf[:D], wf[D:2 * D], wf[2 * D:]
    return pl.pallas_call(
        _final_body,
        grid=(N // RBLK,),
        in_specs=[
            pl.BlockSpec((RBLK, D), lambda i: (i, 0)),
            pl.BlockSpec((RBLK, D), lambda i: (i, 0)),
            pl.BlockSpec((2, RBLK, D), lambda i: (0, i, 0)),
            pl.BlockSpec((2, RBLK, D), lambda i: (0, i, 0)),
            pl.BlockSpec((D, D), lambda i: (0, 0)),
            pl.BlockSpec((1, D), lambda i: (0, 0)),
            pl.BlockSpec((D, D), lambda i: (0, 0)),
            pl.BlockSpec((D, D), lambda i: (0, 0)),
            pl.BlockSpec((D, D), lambda i: (0, 0)),
            pl.BlockSpec((1, D), lambda i: (0, 0)),
        ],
        out_specs=pl.BlockSpec((RBLK, D), lambda i: (i, 0)),
        out_shape=jax.ShapeDtypeStruct((N, D), jnp.float32),
    )(x, x1, sums, cnt_b, wr, br.reshape(1, D), wfa, wfb, wfc,
      bf.reshape(1, D))


def _regroup(sums_flat):
    return (sums_flat.reshape(2, NG, N, 16)
            .transpose(0, 2, 1, 3)
            .reshape(2, N, D))


def _cnt_bcast(cnts_flat):
    return jnp.broadcast_to(cnts_flat.reshape(2, N, 1), (2, N, D))


def kernel(x, edge_index, edge_weight, W1_0, W2_0, Wr_0, br_0,
           W1_1, W2_1, Wr_1, br_1, Wf, bf):
    del edge_weight, W2_0, W2_1
    src = edge_index[0]
    dst = edge_index[1]
    goff = (jnp.arange(NG, dtype=jnp.int32) * N)[None, :, None]
    gidx_exp = (jnp.stack([src, dst])[:, None, :] + goff).reshape(-1)

    lanes = jnp.arange(16, dtype=jnp.int32)
    sidx = jnp.stack([dst, src])[:, None, :]
    lo = (jnp.arange(NH, dtype=jnp.int32) * NLOC)[None, :, None]
    rel = sidx - lo
    inh = (rel >= 0) & (rel < NLOC)
    libase = jnp.where(inh, rel * 16, NLOC * 16)
    li16 = libase[..., None] + lanes
    c0 = jnp.where(inh, rel, NLOC)
    ci16 = jnp.where(lanes == 0, c0[..., None], NLOC + lanes)
    licid = jnp.concatenate(
        [li16.reshape(2, NH, NCHUNK, CH * 16),
         ci16.reshape(2, NH, NCHUNK, CH * 16)], axis=-1).reshape(-1)

    h0c = _tc_matmul_cols(x, W1_0)
    sums0, cnts0 = _sc_segment_sums(h0c.reshape(NG * N, 16), gidx_exp, licid)
    cnt_b = _cnt_bcast(cnts0)
    x1, h1c = _tc_combine(x, _regroup(sums0), cnt_b, Wr_0, br_0, W1_1)
    sums1, _ = _sc_segment_sums(h1c.reshape(NG * N, 16), gidx_exp, licid)
    out = _tc_final(x, x1, _regroup(sums1), cnt_b, Wr_1, br_1, Wf, bf)
    return out
